# baseline (speedup 1.0000x reference)
import functools

import jax
import jax.numpy as jnp
from jax.experimental import pallas as pl
from jax.experimental.pallas import tpu as pltpu


def _router_kernel(x_ref, w1_ref, b1_ref, w2_ref, b2_ref,
                   logits_ref, gate_ref, hacc_ref, lacc_ref):
    k = pl.program_id(1)
    j = pl.program_id(2)
    nk = pl.num_programs(1)
    nj = pl.num_programs(2)
    th = w1_ref.shape[1]
    cols = pl.ds(j * th, th)

    partial = jnp.dot(x_ref[...].astype(jnp.bfloat16), w1_ref[...],
                      preferred_element_type=jnp.float32)

    @pl.when(k == 0)
    def _init_h():
        hacc_ref[:, cols] = partial

    @pl.when((k != 0) & (k != nk - 1))
    def _accum_h():
        hacc_ref[:, cols] += partial

    @pl.when(k == nk - 1)
    def _finish_block():
        prev = jnp.where(nk == 1, 0.0, hacc_ref[:, cols])
        h = prev + partial + b1_ref[...]
        h = h * jax.nn.sigmoid(h)
        p2 = jnp.dot(h.astype(jnp.bfloat16), w2_ref[...],
                     preferred_element_type=jnp.float32)

        @pl.when(j == 0)
        def _init_l():
            lacc_ref[...] = p2

        @pl.when((j != 0) & (j != nj - 1))
        def _accum_l():
            lacc_ref[...] += p2

        @pl.when(j == nj - 1)
        def _epilogue():
            prev_l = jnp.where(nj == 1, 0.0, lacc_ref[...])
            logits = prev_l + p2 + b2_ref[...]
            logits_ref[...] = logits
            m = jnp.max(logits, axis=-1, keepdims=True)
            e = jnp.exp(logits - m)
            gate_ref[...] = e / jnp.sum(e, axis=-1, keepdims=True)


@functools.partial(jax.jit, static_argnames=("tm", "th", "dk"))
def _router(flow_input, W1, b1, W2, b2, tm=1024, th=2048, dk=1024):
    tokens, d_model = flow_input.shape
    hidden, num_experts = W2.shape
    tm = min(tm, tokens)
    th = min(th, hidden)
    dk = min(dk, d_model)
    ni = tokens // tm
    nj = hidden // th
    nk = d_model // dk

    W1 = W1.astype(jnp.bfloat16)
    W2 = W2.astype(jnp.bfloat16)
    b1_2d = b1.reshape(1, hidden)
    b2_2d = b2.reshape(1, num_experts)

    out_shapes = (
        jax.ShapeDtypeStruct((tokens, num_experts), jnp.float32),
        jax.ShapeDtypeStruct((tokens, num_experts), jnp.float32),
    )

    grid_spec = pltpu.PrefetchScalarGridSpec(
        num_scalar_prefetch=0,
        grid=(ni, nk, nj),
        in_specs=[
            pl.BlockSpec((tm, dk), lambda i, k, j: (i, k)),
            pl.BlockSpec((dk, th), lambda i, k, j: (k, j)),
            pl.BlockSpec((1, th), lambda i, k, j: (0, j)),
            pl.BlockSpec((th, num_experts), lambda i, k, j: (j, 0)),
            pl.BlockSpec((1, num_experts), lambda i, k, j: (0, 0)),
        ],
        out_specs=[
            pl.BlockSpec((tm, num_experts), lambda i, k, j: (i, 0)),
            pl.BlockSpec((tm, num_experts), lambda i, k, j: (i, 0)),
        ],
        scratch_shapes=[pltpu.VMEM((tm, hidden), jnp.float32),
                        pltpu.VMEM((tm, num_experts), jnp.float32)],
    )

    return pl.pallas_call(
        _router_kernel,
        grid_spec=grid_spec,
        out_shape=out_shapes,
        compiler_params=pltpu.CompilerParams(
            dimension_semantics=("parallel", "arbitrary", "arbitrary"),
        ),
    )(flow_input, W1, b1_2d, W2, b2_2d)


def kernel(flow_input, W1, b1, W2, b2):
    return _router(flow_input, W1, b1, W2, b2)


# speedup vs baseline: 1.1798x; 1.1798x over previous
import functools

import jax
import jax.numpy as jnp
from jax.experimental import pallas as pl
from jax.experimental.pallas import tpu as pltpu


def _router_kernel(x_ref, w1_ref, b1_ref, w2_ref, b2_ref,
                   logits_ref, gate_ref, acc_ref):
    j = pl.program_id(1)
    nj = pl.num_programs(1)

    h = jnp.dot(x_ref[...].astype(jnp.bfloat16), w1_ref[...],
                preferred_element_type=jnp.float32)
    h = h + b1_ref[...]
    h = h * jax.nn.sigmoid(h)
    part = jnp.dot(h.astype(jnp.bfloat16), w2_ref[...],
                   preferred_element_type=jnp.float32)

    @pl.when(j == 0)
    def _init():
        acc_ref[...] = part

    @pl.when((j != 0) & (j != nj - 1))
    def _accum():
        acc_ref[...] += part

    @pl.when(j == nj - 1)
    def _finish():
        prev = jnp.where(nj == 1, 0.0, acc_ref[...])
        logits = prev + part + b2_ref[...]
        logits_ref[...] = logits
        m = jnp.max(logits, axis=-1, keepdims=True)
        e = jnp.exp(logits - m)
        gate_ref[...] = e / jnp.sum(e, axis=-1, keepdims=True)


@functools.partial(jax.jit, static_argnames=("tm", "th"))
def _router(flow_input, W1, b1, W2, b2, tm=512, th=4096):
    tokens, d_model = flow_input.shape
    hidden, num_experts = W2.shape
    tm = min(tm, tokens)
    th = min(th, hidden)
    ni = tokens // tm
    nj = hidden // th

    W1 = W1.astype(jnp.bfloat16)
    W2 = W2.astype(jnp.bfloat16)
    b1_2d = b1.reshape(1, hidden)
    b2_2d = b2.reshape(1, num_experts)

    out_shapes = (
        jax.ShapeDtypeStruct((tokens, num_experts), jnp.float32),
        jax.ShapeDtypeStruct((tokens, num_experts), jnp.float32),
    )

    grid_spec = pltpu.PrefetchScalarGridSpec(
        num_scalar_prefetch=0,
        grid=(ni, nj),
        in_specs=[
            pl.BlockSpec((tm, d_model), lambda i, j: (i, 0)),
            pl.BlockSpec((d_model, th), lambda i, j: (0, j)),
            pl.BlockSpec((1, th), lambda i, j: (0, j)),
            pl.BlockSpec((th, num_experts), lambda i, j: (j, 0)),
            pl.BlockSpec((1, num_experts), lambda i, j: (0, 0)),
        ],
        out_specs=[
            pl.BlockSpec((tm, num_experts), lambda i, j: (i, 0)),
            pl.BlockSpec((tm, num_experts), lambda i, j: (i, 0)),
        ],
        scratch_shapes=[pltpu.VMEM((tm, num_experts), jnp.float32)],
    )

    return pl.pallas_call(
        _router_kernel,
        grid_spec=grid_spec,
        out_shape=out_shapes,
        compiler_params=pltpu.CompilerParams(
            dimension_semantics=("parallel", "arbitrary"),
            vmem_limit_bytes=62 * 1024 * 1024,
        ),
    )(flow_input, W1, b1_2d, W2, b2_2d)


def kernel(flow_input, W1, b1, W2, b2):
    return _router(flow_input, W1, b1, W2, b2)
